# baseline (speedup 1.0000x reference)
import functools

import jax
import jax.numpy as jnp
from jax import lax
from jax.experimental import pallas as pl
from jax.experimental.pallas import tpu as pltpu
from jax.experimental.pallas import tpu_sc as plsc

NC = 2
NS = 16
NW = NC * NS
LANES = 16
CHUNK = 128


@functools.cache
def _make_agg(NP: int, D: int, CC: int, CHM: int, CH0: int, CH1: int,
              gather: bool):
    Z = NP // NS
    ZC, ZR = divmod(Z, CC)

    mesh = plsc.VectorSubcoreMesh(core_axis_name="c", subcore_axis_name="s")

    def body(*refs):
        if gather:
            table, srcw, dstw, out, src_v, dst_v, buf, acc, gsem = refs
        else:
            dstw, out, dst_v, buf, acc = refs
        c = lax.axis_index("c")
        s = lax.axis_index("s")
        w = c * NS + s

        def fill(val):
            def frow(i, carry):
                for k in range(D // LANES):
                    buf[i, pl.ds(k * LANES, LANES)] = jnp.full(
                        (LANES,), val, jnp.float32)
                return carry
            lax.fori_loop(0, CC, frow, 0)

        fill(0.0)
        for z in range(ZC):
            pltpu.sync_copy(buf, acc.at[pl.ds(s * Z + z * CC, CC)])
        if ZR:
            pltpu.sync_copy(buf.at[pl.ds(0, ZR)],
                            acc.at[pl.ds(s * Z + ZC * CC, ZR)])
        if not gather:
            fill(1.0)

        pltpu.sync_copy(dstw.at[w], dst_v)
        if gather:
            pltpu.sync_copy(srcw.at[w], src_v)

        plsc.subcore_barrier()

        nch = jnp.where(c == 0, CH0, CH1)
        if gather:
            def chunk_g(j, cr):
                pltpu.async_copy(table.at[src_v.at[j]], buf, gsem).wait()
                pltpu.sync_copy(buf, acc.at[dst_v.at[j]], add=True)
                return cr
            lax.fori_loop(0, nch, chunk_g, 0)
        else:
            def chunk_s(j, cr):
                pltpu.sync_copy(buf, acc.at[dst_v.at[j]], add=True)
                return cr
            lax.fori_loop(0, nch, chunk_s, 0)

        plsc.subcore_barrier()

        pltpu.sync_copy(acc.at[pl.ds(s * Z, Z)],
                        out.at[pl.ds(c * NP + s * Z, Z)])

    scratch = []
    if gather:
        scratch.append(pltpu.VMEM((CHM, CC), jnp.int32))
    scratch.append(pltpu.VMEM((CHM, CC), jnp.int32))
    scratch.append(pltpu.VMEM((CC, D), jnp.float32))
    scratch.append(pltpu.VMEM_SHARED((NP, D), jnp.float32))
    if gather:
        scratch.append(pltpu.SemaphoreType.DMA)
    params = None
    if D % 128 != 0:
        params = pltpu.CompilerParams(use_tc_tiling_on_sc=False)
    return pl.kernel(
        body,
        out_type=jax.ShapeDtypeStruct((NC * NP, D), jnp.float32),
        mesh=mesh,
        scratch_types=scratch,
        compiler_params=params,
    )


def _row_spec(rows, cols):
    return pl.BlockSpec((rows, cols), lambda i: (i, 0))


def _full_spec(r, c):
    return pl.BlockSpec((r, c), lambda i: (0, 0))


def _prep_body(d0, d1, x, n_ref, u_ref):
    deg = d0[:, 0:1] + d1[:, 0:1]
    nm = lax.rsqrt(jnp.maximum(deg, 1.0))
    n_ref[...] = nm
    u_ref[...] = x[...] * nm


def _prep(NP, B, d0, d1, xp):
    return pl.pallas_call(
        lambda a, b, c, d, e: _prep_body(a, b, c, d, e),
        grid=(NP // B,),
        in_specs=[_row_spec(B, 16), _row_spec(B, 16), _row_spec(B, 128)],
        out_specs=[_row_spec(B, 1), _row_spec(B, 128)],
        out_shape=[jax.ShapeDtypeStruct((NP, 1), jnp.float32),
                   jax.ShapeDtypeStruct((NP, 128), jnp.float32)],
    )(d0, d1, xp)


def _layer1_body(p0, p1, n, x, w, h_ref, u_ref):
    s = (p0[...] + p1[...]) * n[...]
    h = jnp.tanh(jnp.dot(s, w[...], preferred_element_type=jnp.float32)
                 + x[...])
    h_ref[...] = h
    u_ref[...] = h * n[...]


def _layer1(NP, B, p0, p1, n, xp, w1b):
    return pl.pallas_call(
        _layer1_body,
        grid=(NP // B,),
        in_specs=[_row_spec(B, 128), _row_spec(B, 128), _row_spec(B, 1),
                  _row_spec(B, 128), _full_spec(128, 128)],
        out_specs=[_row_spec(B, 128), _row_spec(B, 128)],
        out_shape=[jax.ShapeDtypeStruct((NP, 128), jnp.float32),
                   jax.ShapeDtypeStruct((NP, 128), jnp.float32)],
    )(p0, p1, n, xp, w1b)


def _layer2_body(p0, p1, n, h1, w2, w3, z_ref):
    s = (p0[...] + p1[...]) * n[...]
    h2 = jnp.tanh(jnp.dot(s, w2[...], preferred_element_type=jnp.float32)
                  + h1[...])
    z_ref[...] = jnp.dot(h2 * n[...], w3[...],
                         preferred_element_type=jnp.float32)


def _layer2(NP, B, p0, p1, n, h1, w2b, w3b):
    return pl.pallas_call(
        _layer2_body,
        grid=(NP // B,),
        in_specs=[_row_spec(B, 128), _row_spec(B, 128), _row_spec(B, 1),
                  _row_spec(B, 128), _full_spec(128, 128),
                  _full_spec(128, 64)],
        out_specs=_row_spec(B, 64),
        out_shape=jax.ShapeDtypeStruct((NP, 64), jnp.float32),
    )(p0, p1, n, h1, w2b, w3b)


def _final_body(q0, q1, n, o_ref):
    o_ref[...] = (q0[...] + q1[...]) * n[...]


def _final(NP, B, q0, q1, n):
    return pl.pallas_call(
        _final_body,
        grid=(NP // B,),
        in_specs=[_row_spec(B, 64), _row_spec(B, 64), _row_spec(B, 1)],
        out_specs=_row_spec(B, 64),
        out_shape=jax.ShapeDtypeStruct((NP, 64), jnp.float32),
    )(q0, q1, n)


def kernel(x, edge_index, W1, W2, W3, beta1, beta2, beta3):
    N, D = x.shape
    E = edge_index.shape[1]
    DO = W3.shape[1]

    B = 1024
    NP = -(-(N + 1) // (NS * CHUNK)) * (NS * CHUNK)

    src = edge_index[0]
    dst = edge_index[1]

    CC = CHUNK
    CHT = (E + NS * CC - 1) // (NS * CC)
    CH0 = CHT // 3
    CH1 = CHT - CH0
    CHM = max(CH0, CH1)

    def edge_layout(idx, fillval):
        ep = NS * CHT * CC
        flat = jnp.pad(idx, (0, ep - E), constant_values=fillval)
        blk = flat.reshape(NS, CHT, CC)
        c0 = blk[:, :CH0]
        c1 = blk[:, CH0:]
        pad0 = jnp.full((NS, CHM - CH0, CC), fillval, jnp.int32)
        pad1 = jnp.full((NS, CHM - CH1, CC), fillval, jnp.int32)
        return jnp.concatenate(
            [jnp.concatenate([c0, pad0], axis=1),
             jnp.concatenate([c1, pad1], axis=1)], axis=0)

    srcw_e = edge_layout(src, 0)
    dstw_e = edge_layout(dst, N)
    xp = jnp.pad(x, ((0, NP - N), (0, 0)))

    w1b = W1 * beta1
    w2b = W2 * beta2
    w3b = W3 * beta3

    degp = _make_agg(NP, 16, CC, CHM, CH0, CH1, False)(dstw_e)
    norm, u1 = _prep(NP, B, degp[:NP], degp[NP:], xp)

    agg_d = _make_agg(NP, D, CC, CHM, CH0, CH1, True)
    s1 = agg_d(u1, srcw_e, dstw_e)
    h1, u2 = _layer1(NP, B, s1[:NP], s1[NP:], norm, xp, w1b)

    s2 = agg_d(u2, srcw_e, dstw_e)
    z3 = _layer2(NP, B, s2[:NP], s2[NP:], norm, h1, w2b, w3b)

    s3 = _make_agg(NP, DO, CC, CHM, CH0, CH1, True)(z3, srcw_e, dstw_e)
    out = _final(NP, B, s3[:NP], s3[NP:], norm)
    return out[:N]


# speedup vs baseline: 1.1359x; 1.1359x over previous
import functools

import jax
import jax.numpy as jnp
from jax import lax
from jax.experimental import pallas as pl
from jax.experimental.pallas import tpu as pltpu
from jax.experimental.pallas import tpu_sc as plsc

NC = 2
NS = 16
NW = NC * NS
LANES = 16
CHUNK = 128


@functools.cache
def _make_agg(NP: int, D: int, CC: int, CHM: int, CH0: int, CH1: int,
              gather: bool):
    Z = NP // NS
    ZC, ZR = divmod(Z, CC)

    mesh = plsc.VectorSubcoreMesh(core_axis_name="c", subcore_axis_name="s")

    def body(*refs):
        if gather:
            table, srcw, dstw, out, src_v, dst_v, buf, acc, gsem = refs
        else:
            dstw, out, dst_v, buf, acc = refs
        c = lax.axis_index("c")
        s = lax.axis_index("s")
        w = c * NS + s

        def fill(val):
            def frow(i, carry):
                for k in range(D // LANES):
                    buf[i, pl.ds(k * LANES, LANES)] = jnp.full(
                        (LANES,), val, jnp.float32)
                return carry
            lax.fori_loop(0, CC, frow, 0)

        fill(0.0)
        for z in range(ZC):
            pltpu.sync_copy(buf, acc.at[pl.ds(s * Z + z * CC, CC)])
        if ZR:
            pltpu.sync_copy(buf.at[pl.ds(0, ZR)],
                            acc.at[pl.ds(s * Z + ZC * CC, ZR)])
        if not gather:
            fill(1.0)

        pltpu.sync_copy(dstw.at[w], dst_v)
        if gather:
            pltpu.sync_copy(srcw.at[w], src_v)

        plsc.subcore_barrier()

        nch = jnp.where(c == 0, CH0, CH1)
        if gather:
            def chunk_g(j, cr):
                pltpu.async_copy(table.at[src_v.at[j]], buf, gsem).wait()
                pltpu.sync_copy(buf, acc.at[dst_v.at[j]], add=True)
                return cr
            lax.fori_loop(0, nch, chunk_g, 0)
        else:
            def chunk_s(j, cr):
                pltpu.sync_copy(buf, acc.at[dst_v.at[j]], add=True)
                return cr
            lax.fori_loop(0, nch, chunk_s, 0)

        plsc.subcore_barrier()

        pltpu.sync_copy(acc.at[pl.ds(s * Z, Z)],
                        out.at[pl.ds(c * NP + s * Z, Z)])

    scratch = []
    if gather:
        scratch.append(pltpu.VMEM((CHM, CC), jnp.int32))
    scratch.append(pltpu.VMEM((CHM, CC), jnp.int32))
    scratch.append(pltpu.VMEM((CC, D), jnp.float32))
    scratch.append(pltpu.VMEM_SHARED((NP, D), jnp.float32))
    if gather:
        scratch.append(pltpu.SemaphoreType.DMA)
    params = None
    if D % 128 != 0:
        params = pltpu.CompilerParams(use_tc_tiling_on_sc=False)
    return pl.kernel(
        body,
        out_type=jax.ShapeDtypeStruct((NC * NP, D), jnp.float32),
        mesh=mesh,
        scratch_types=scratch,
        compiler_params=params,
    )


def _row_spec(rows, cols):
    return pl.BlockSpec((rows, cols), lambda i: (i, 0))


def _full_spec(r, c):
    return pl.BlockSpec((r, c), lambda i: (0, 0))


def _prep_body(d0, d1, x, n_ref, u_ref):
    deg = d0[:, 0:1] + d1[:, 0:1]
    nm = lax.rsqrt(jnp.maximum(deg, 1.0))
    n_ref[...] = nm
    u_ref[...] = x[...] * nm


def _prep(NP, B, d0, d1, xp):
    return pl.pallas_call(
        lambda a, b, c, d, e: _prep_body(a, b, c, d, e),
        grid=(NP // B,),
        in_specs=[_row_spec(B, 16), _row_spec(B, 16), _row_spec(B, 128)],
        out_specs=[_row_spec(B, 1), _row_spec(B, 128)],
        out_shape=[jax.ShapeDtypeStruct((NP, 1), jnp.float32),
                   jax.ShapeDtypeStruct((NP, 128), jnp.float32)],
    )(d0, d1, xp)


def _layer1_body(p0, p1, n, x, w, h_ref, u_ref):
    s = (p0[...] + p1[...]) * n[...]
    h = jnp.tanh(jnp.dot(s, w[...], preferred_element_type=jnp.float32)
                 + x[...])
    h_ref[...] = h
    u_ref[...] = h * n[...]


def _layer1(NP, B, p0, p1, n, xp, w1b):
    return pl.pallas_call(
        _layer1_body,
        grid=(NP // B,),
        in_specs=[_row_spec(B, 128), _row_spec(B, 128), _row_spec(B, 1),
                  _row_spec(B, 128), _full_spec(128, 128)],
        out_specs=[_row_spec(B, 128), _row_spec(B, 128)],
        out_shape=[jax.ShapeDtypeStruct((NP, 128), jnp.float32),
                   jax.ShapeDtypeStruct((NP, 128), jnp.float32)],
    )(p0, p1, n, xp, w1b)


def _layer2_body(p0, p1, n, h1, w2, w3, z_ref):
    s = (p0[...] + p1[...]) * n[...]
    h2 = jnp.tanh(jnp.dot(s, w2[...], preferred_element_type=jnp.float32)
                  + h1[...])
    z_ref[...] = jnp.dot(h2 * n[...], w3[...],
                         preferred_element_type=jnp.float32)


def _layer2(NP, B, p0, p1, n, h1, w2b, w3b):
    return pl.pallas_call(
        _layer2_body,
        grid=(NP // B,),
        in_specs=[_row_spec(B, 128), _row_spec(B, 128), _row_spec(B, 1),
                  _row_spec(B, 128), _full_spec(128, 128),
                  _full_spec(128, 64)],
        out_specs=_row_spec(B, 64),
        out_shape=jax.ShapeDtypeStruct((NP, 64), jnp.float32),
    )(p0, p1, n, h1, w2b, w3b)


def _final_body(q0, q1, n, o_ref):
    o_ref[...] = (q0[...] + q1[...]) * n[...]


def _final(NP, B, q0, q1, n):
    return pl.pallas_call(
        _final_body,
        grid=(NP // B,),
        in_specs=[_row_spec(B, 64), _row_spec(B, 64), _row_spec(B, 1)],
        out_specs=_row_spec(B, 64),
        out_shape=jax.ShapeDtypeStruct((NP, 64), jnp.float32),
    )(q0, q1, n)


def kernel(x, edge_index, W1, W2, W3, beta1, beta2, beta3):
    N, D = x.shape
    E = edge_index.shape[1]
    DO = W3.shape[1]

    B = 1024
    NP = -(-(N + 1) // (NS * CHUNK)) * (NS * CHUNK)

    src = edge_index[0]
    dst = edge_index[1]

    CC = CHUNK
    CHT = (E + NS * CC - 1) // (NS * CC)
    CH1 = CHT // 3
    CH0 = CHT - CH1
    CHM = max(CH0, CH1)

    def edge_layout(idx, fillval):
        ep = NS * CHT * CC
        flat = jnp.pad(idx, (0, ep - E), constant_values=fillval)
        blk = flat.reshape(NS, CHT, CC)
        c0 = blk[:, :CH0]
        c1 = blk[:, CH0:]
        pad0 = jnp.full((NS, CHM - CH0, CC), fillval, jnp.int32)
        pad1 = jnp.full((NS, CHM - CH1, CC), fillval, jnp.int32)
        return jnp.concatenate(
            [jnp.concatenate([c0, pad0], axis=1),
             jnp.concatenate([c1, pad1], axis=1)], axis=0)

    srcw_e = edge_layout(src, 0)
    dstw_e = edge_layout(dst, N)
    xp = jnp.pad(x, ((0, NP - N), (0, 0)))

    w1b = W1 * beta1
    w2b = W2 * beta2
    w3b = W3 * beta3

    degp = _make_agg(NP, 16, CC, CHM, CH0, CH1, False)(dstw_e)
    norm, u1 = _prep(NP, B, degp[:NP], degp[NP:], xp)

    agg_d = _make_agg(NP, D, CC, CHM, CH0, CH1, True)
    s1 = agg_d(u1, srcw_e, dstw_e)
    h1, u2 = _layer1(NP, B, s1[:NP], s1[NP:], norm, xp, w1b)

    s2 = agg_d(u2, srcw_e, dstw_e)
    z3 = _layer2(NP, B, s2[:NP], s2[NP:], norm, h1, w2b, w3b)

    s3 = _make_agg(NP, DO, CC, CHM, CH0, CH1, True)(z3, srcw_e, dstw_e)
    out = _final(NP, B, s3[:NP], s3[NP:], norm)
    return out[:N]


# speedup vs baseline: 1.1888x; 1.0465x over previous
import functools

import jax
import jax.numpy as jnp
from jax import lax
from jax.experimental import pallas as pl
from jax.experimental.pallas import tpu as pltpu
from jax.experimental.pallas import tpu_sc as plsc

NC = 2
NS = 16
NW = NC * NS
LANES = 16
CHUNK = 128


@functools.cache
def _make_agg(NP: int, D: int, CC: int, CHM: int, CH0: int, CH1: int,
              gather: bool):
    Z = NP // NS
    ZC, ZR = divmod(Z, CC)

    mesh = plsc.VectorSubcoreMesh(core_axis_name="c", subcore_axis_name="s")

    def body(*refs):
        if gather:
            table, srcw, dstw, out, src_v, dst_v, buf, acc, gsem = refs
        else:
            dstw, out, dst_v, buf, acc = refs
        c = lax.axis_index("c")
        s = lax.axis_index("s")
        w = c * NS + s

        def fill(val):
            def frow(i, carry):
                for k in range(D // LANES):
                    buf[i, pl.ds(k * LANES, LANES)] = jnp.full(
                        (LANES,), val, jnp.float32)
                return carry
            lax.fori_loop(0, CC, frow, 0)

        fill(0.0)
        for z in range(ZC):
            pltpu.sync_copy(buf, acc.at[pl.ds(s * Z + z * CC, CC)])
        if ZR:
            pltpu.sync_copy(buf.at[pl.ds(0, ZR)],
                            acc.at[pl.ds(s * Z + ZC * CC, ZR)])
        if not gather:
            fill(1.0)

        pltpu.sync_copy(dstw.at[w], dst_v)
        if gather:
            pltpu.sync_copy(srcw.at[w], src_v)

        plsc.subcore_barrier()

        nch = jnp.where(c == 0, CH0, CH1)
        if gather:
            def chunk_g(j, cr):
                pltpu.async_copy(table.at[src_v.at[j]], buf, gsem).wait()
                pltpu.sync_copy(buf, acc.at[dst_v.at[j]], add=True)
                return cr
            lax.fori_loop(0, nch, chunk_g, 0)
        else:
            def chunk_s(j, cr):
                pltpu.sync_copy(buf, acc.at[dst_v.at[j]], add=True)
                return cr
            lax.fori_loop(0, nch, chunk_s, 0)

        plsc.subcore_barrier()

        pltpu.sync_copy(acc.at[pl.ds(s * Z, Z)],
                        out.at[pl.ds(c * NP + s * Z, Z)])

    scratch = []
    if gather:
        scratch.append(pltpu.VMEM((CHM, CC), jnp.int32))
    scratch.append(pltpu.VMEM((CHM, CC), jnp.int32))
    scratch.append(pltpu.VMEM((CC, D), jnp.float32))
    scratch.append(pltpu.VMEM_SHARED((NP, D), jnp.float32))
    if gather:
        scratch.append(pltpu.SemaphoreType.DMA)
    params = None
    if D % 128 != 0:
        params = pltpu.CompilerParams(use_tc_tiling_on_sc=False)
    return pl.kernel(
        body,
        out_type=jax.ShapeDtypeStruct((NC * NP, D), jnp.float32),
        mesh=mesh,
        scratch_types=scratch,
        compiler_params=params,
    )


def _row_spec(rows, cols):
    return pl.BlockSpec((rows, cols), lambda i: (i, 0))


def _full_spec(r, c):
    return pl.BlockSpec((r, c), lambda i: (0, 0))


def _prep_body(d0, d1, x, n_ref, u_ref):
    deg = d0[:, 0:1] + d1[:, 0:1]
    nm = lax.rsqrt(jnp.maximum(deg, 1.0))
    n_ref[...] = nm
    u_ref[...] = x[...] * nm


def _prep(NP, B, d0, d1, xp):
    return pl.pallas_call(
        lambda a, b, c, d, e: _prep_body(a, b, c, d, e),
        grid=(NP // B,),
        in_specs=[_row_spec(B, 16), _row_spec(B, 16), _row_spec(B, 128)],
        out_specs=[_row_spec(B, 1), _row_spec(B, 128)],
        out_shape=[jax.ShapeDtypeStruct((NP, 1), jnp.float32),
                   jax.ShapeDtypeStruct((NP, 128), jnp.float32)],
    )(d0, d1, xp)


def _layer1_body(p0, p1, n, x, w, h_ref, u_ref):
    s = (p0[...] + p1[...]) * n[...]
    h = jnp.tanh(jnp.dot(s, w[...], preferred_element_type=jnp.float32)
                 + x[...])
    h_ref[...] = h
    u_ref[...] = h * n[...]


def _layer1(NP, B, p0, p1, n, xp, w1b):
    return pl.pallas_call(
        _layer1_body,
        grid=(NP // B,),
        in_specs=[_row_spec(B, 128), _row_spec(B, 128), _row_spec(B, 1),
                  _row_spec(B, 128), _full_spec(128, 128)],
        out_specs=[_row_spec(B, 128), _row_spec(B, 128)],
        out_shape=[jax.ShapeDtypeStruct((NP, 128), jnp.float32),
                   jax.ShapeDtypeStruct((NP, 128), jnp.float32)],
    )(p0, p1, n, xp, w1b)


def _layer2_body(p0, p1, n, h1, w2, w3, z_ref):
    s = (p0[...] + p1[...]) * n[...]
    h2 = jnp.tanh(jnp.dot(s, w2[...], preferred_element_type=jnp.float32)
                  + h1[...])
    z_ref[...] = jnp.dot(h2 * n[...], w3[...],
                         preferred_element_type=jnp.float32)


def _layer2(NP, B, p0, p1, n, h1, w2b, w3b):
    return pl.pallas_call(
        _layer2_body,
        grid=(NP // B,),
        in_specs=[_row_spec(B, 128), _row_spec(B, 128), _row_spec(B, 1),
                  _row_spec(B, 128), _full_spec(128, 128),
                  _full_spec(128, 64)],
        out_specs=_row_spec(B, 64),
        out_shape=jax.ShapeDtypeStruct((NP, 64), jnp.float32),
    )(p0, p1, n, h1, w2b, w3b)


def _final_body(q0, q1, n, o_ref):
    o_ref[...] = (q0[...] + q1[...]) * n[...]


def _final(NP, B, q0, q1, n):
    return pl.pallas_call(
        _final_body,
        grid=(NP // B,),
        in_specs=[_row_spec(B, 64), _row_spec(B, 64), _row_spec(B, 1)],
        out_specs=_row_spec(B, 64),
        out_shape=jax.ShapeDtypeStruct((NP, 64), jnp.float32),
    )(q0, q1, n)


def kernel(x, edge_index, W1, W2, W3, beta1, beta2, beta3):
    N, D = x.shape
    E = edge_index.shape[1]
    DO = W3.shape[1]

    B = 1024
    NP = -(-(N + 1) // (NS * CHUNK)) * (NS * CHUNK)

    src = edge_index[0]
    dst = edge_index[1]

    CC = CHUNK
    CHT = (E + NS * CC - 1) // (NS * CC)
    CH0 = CHT * 3 // 5
    CH1 = CHT - CH0
    CHM = max(CH0, CH1)

    def edge_layout(idx, fillval):
        ep = NS * CHT * CC
        flat = jnp.pad(idx, (0, ep - E), constant_values=fillval)
        blk = flat.reshape(NS, CHT, CC)
        c0 = blk[:, :CH0]
        c1 = blk[:, CH0:]
        pad0 = jnp.full((NS, CHM - CH0, CC), fillval, jnp.int32)
        pad1 = jnp.full((NS, CHM - CH1, CC), fillval, jnp.int32)
        return jnp.concatenate(
            [jnp.concatenate([c0, pad0], axis=1),
             jnp.concatenate([c1, pad1], axis=1)], axis=0)

    srcw_e = edge_layout(src, 0)
    dstw_e = edge_layout(dst, N)
    xp = jnp.pad(x, ((0, NP - N), (0, 0)))

    w1b = W1 * beta1
    w2b = W2 * beta2
    w3b = W3 * beta3

    degp = _make_agg(NP, 16, CC, CHM, CH0, CH1, False)(dstw_e)
    norm, u1 = _prep(NP, B, degp[:NP], degp[NP:], xp)

    agg_d = _make_agg(NP, D, CC, CHM, CH0, CH1, True)
    s1 = agg_d(u1, srcw_e, dstw_e)
    h1, u2 = _layer1(NP, B, s1[:NP], s1[NP:], norm, xp, w1b)

    s2 = agg_d(u2, srcw_e, dstw_e)
    z3 = _layer2(NP, B, s2[:NP], s2[NP:], norm, h1, w2b, w3b)

    s3 = _make_agg(NP, DO, CC, CHM, CH0, CH1, True)(z3, srcw_e, dstw_e)
    out = _final(NP, B, s3[:NP], s3[NP:], norm)
    return out[:N]


# speedup vs baseline: 1.1930x; 1.0035x over previous
import functools

import jax
import jax.numpy as jnp
from jax import lax
from jax.experimental import pallas as pl
from jax.experimental.pallas import tpu as pltpu
from jax.experimental.pallas import tpu_sc as plsc

NC = 2
NS = 16
NW = NC * NS
LANES = 16
CHUNK = 128


@functools.cache
def _make_agg(NP: int, D: int, CC: int, CHM: int, CH0: int, CH1: int,
              gather: bool):
    Z = NP // NS
    ZC, ZR = divmod(Z, CC)

    mesh = plsc.VectorSubcoreMesh(core_axis_name="c", subcore_axis_name="s")

    def body(*refs):
        if gather:
            table, srcw, dstw, out, src_v, dst_v, buf, acc, gsem = refs
        else:
            dstw, out, dst_v, buf, acc = refs
        c = lax.axis_index("c")
        s = lax.axis_index("s")
        w = c * NS + s

        def fill(val):
            def frow(i, carry):
                for k in range(D // LANES):
                    buf[i, pl.ds(k * LANES, LANES)] = jnp.full(
                        (LANES,), val, jnp.float32)
                return carry
            lax.fori_loop(0, CC, frow, 0)

        fill(0.0)
        for z in range(ZC):
            pltpu.sync_copy(buf, acc.at[pl.ds(s * Z + z * CC, CC)])
        if ZR:
            pltpu.sync_copy(buf.at[pl.ds(0, ZR)],
                            acc.at[pl.ds(s * Z + ZC * CC, ZR)])
        if not gather:
            fill(1.0)

        pltpu.sync_copy(dstw.at[w], dst_v)
        if gather:
            pltpu.sync_copy(srcw.at[w], src_v)

        plsc.subcore_barrier()

        nch = jnp.where(c == 0, CH0, CH1)
        if gather:
            def chunk_g(j, cr):
                pltpu.async_copy(table.at[src_v.at[j]], buf, gsem).wait()
                pltpu.sync_copy(buf, acc.at[dst_v.at[j]], add=True)
                return cr
            lax.fori_loop(0, nch, chunk_g, 0)
        else:
            def chunk_s(j, cr):
                pltpu.sync_copy(buf, acc.at[dst_v.at[j]], add=True)
                return cr
            lax.fori_loop(0, nch, chunk_s, 0)

        plsc.subcore_barrier()

        pltpu.sync_copy(acc.at[pl.ds(s * Z, Z)],
                        out.at[pl.ds(c * NP + s * Z, Z)])

    scratch = []
    if gather:
        scratch.append(pltpu.VMEM((CHM, CC), jnp.int32))
    scratch.append(pltpu.VMEM((CHM, CC), jnp.int32))
    scratch.append(pltpu.VMEM((CC, D), jnp.float32))
    scratch.append(pltpu.VMEM_SHARED((NP, D), jnp.float32))
    if gather:
        scratch.append(pltpu.SemaphoreType.DMA)
    params = None
    if D % 128 != 0:
        params = pltpu.CompilerParams(use_tc_tiling_on_sc=False)
    return pl.kernel(
        body,
        out_type=jax.ShapeDtypeStruct((NC * NP, D), jnp.float32),
        mesh=mesh,
        scratch_types=scratch,
        compiler_params=params,
    )


def _row_spec(rows, cols):
    return pl.BlockSpec((rows, cols), lambda i: (i, 0))


def _full_spec(r, c):
    return pl.BlockSpec((r, c), lambda i: (0, 0))


def _prep_body(d0, d1, x, n_ref, u_ref):
    deg = d0[:, 0:1] + d1[:, 0:1]
    nm = lax.rsqrt(jnp.maximum(deg, 1.0))
    n_ref[...] = nm
    u_ref[...] = x[...] * nm


def _prep(NP, B, d0, d1, xp):
    return pl.pallas_call(
        lambda a, b, c, d, e: _prep_body(a, b, c, d, e),
        grid=(NP // B,),
        in_specs=[_row_spec(B, 16), _row_spec(B, 16), _row_spec(B, 128)],
        out_specs=[_row_spec(B, 1), _row_spec(B, 128)],
        out_shape=[jax.ShapeDtypeStruct((NP, 1), jnp.float32),
                   jax.ShapeDtypeStruct((NP, 128), jnp.float32)],
    )(d0, d1, xp)


def _layer1_body(p0, p1, n, x, w, h_ref, u_ref):
    s = (p0[...] + p1[...]) * n[...]
    h = jnp.tanh(jnp.dot(s, w[...], preferred_element_type=jnp.float32)
                 + x[...])
    h_ref[...] = h
    u_ref[...] = h * n[...]


def _layer1(NP, B, p0, p1, n, xp, w1b):
    return pl.pallas_call(
        _layer1_body,
        grid=(NP // B,),
        in_specs=[_row_spec(B, 128), _row_spec(B, 128), _row_spec(B, 1),
                  _row_spec(B, 128), _full_spec(128, 128)],
        out_specs=[_row_spec(B, 128), _row_spec(B, 128)],
        out_shape=[jax.ShapeDtypeStruct((NP, 128), jnp.float32),
                   jax.ShapeDtypeStruct((NP, 128), jnp.float32)],
    )(p0, p1, n, xp, w1b)


def _layer2_body(p0, p1, n, h1, w2, w3, z_ref):
    s = (p0[...] + p1[...]) * n[...]
    h2 = jnp.tanh(jnp.dot(s, w2[...], preferred_element_type=jnp.float32)
                  + h1[...])
    z_ref[...] = jnp.dot(h2 * n[...], w3[...],
                         preferred_element_type=jnp.float32)


def _layer2(NP, B, p0, p1, n, h1, w2b, w3b):
    return pl.pallas_call(
        _layer2_body,
        grid=(NP // B,),
        in_specs=[_row_spec(B, 128), _row_spec(B, 128), _row_spec(B, 1),
                  _row_spec(B, 128), _full_spec(128, 128),
                  _full_spec(128, 64)],
        out_specs=_row_spec(B, 64),
        out_shape=jax.ShapeDtypeStruct((NP, 64), jnp.float32),
    )(p0, p1, n, h1, w2b, w3b)


def _final_body(q0, q1, n, o_ref):
    o_ref[...] = (q0[...] + q1[...]) * n[...]


def _final(NP, B, q0, q1, n):
    return pl.pallas_call(
        _final_body,
        grid=(NP // B,),
        in_specs=[_row_spec(B, 64), _row_spec(B, 64), _row_spec(B, 1)],
        out_specs=_row_spec(B, 64),
        out_shape=jax.ShapeDtypeStruct((NP, 64), jnp.float32),
    )(q0, q1, n)


def kernel(x, edge_index, W1, W2, W3, beta1, beta2, beta3):
    N, D = x.shape
    E = edge_index.shape[1]
    DO = W3.shape[1]

    B = 1024
    NP = -(-(N + 1) // (NS * CHUNK)) * (NS * CHUNK)

    src = edge_index[0]
    dst = edge_index[1]

    CC = CHUNK
    CHT = (E + NS * CC - 1) // (NS * CC)
    CH0 = CHT * 5 // 8
    CH1 = CHT - CH0
    CHM = max(CH0, CH1)

    def edge_layout(idx, fillval):
        ep = NS * CHT * CC
        flat = jnp.pad(idx, (0, ep - E), constant_values=fillval)
        blk = flat.reshape(NS, CHT, CC)
        c0 = blk[:, :CH0]
        c1 = blk[:, CH0:]
        pad0 = jnp.full((NS, CHM - CH0, CC), fillval, jnp.int32)
        pad1 = jnp.full((NS, CHM - CH1, CC), fillval, jnp.int32)
        return jnp.concatenate(
            [jnp.concatenate([c0, pad0], axis=1),
             jnp.concatenate([c1, pad1], axis=1)], axis=0)

    srcw_e = edge_layout(src, 0)
    dstw_e = edge_layout(dst, N)
    xp = jnp.pad(x, ((0, NP - N), (0, 0)))

    w1b = W1 * beta1
    w2b = W2 * beta2
    w3b = W3 * beta3

    degp = _make_agg(NP, 16, CC, CHM, CH0, CH1, False)(dstw_e)
    norm, u1 = _prep(NP, B, degp[:NP], degp[NP:], xp)

    agg_d = _make_agg(NP, D, CC, CHM, CH0, CH1, True)
    s1 = agg_d(u1, srcw_e, dstw_e)
    h1, u2 = _layer1(NP, B, s1[:NP], s1[NP:], norm, xp, w1b)

    s2 = agg_d(u2, srcw_e, dstw_e)
    z3 = _layer2(NP, B, s2[:NP], s2[NP:], norm, h1, w2b, w3b)

    s3 = _make_agg(NP, DO, CC, CHM, CH0, CH1, True)(z3, srcw_e, dstw_e)
    out = _final(NP, B, s3[:NP], s3[NP:], norm)
    return out[:N]
